# baseline (speedup 1.0000x reference)
import functools

import jax
import jax.numpy as jnp
from jax import lax
from jax.experimental import pallas as pl
from jax.experimental.pallas import tpu as pltpu
from jax.experimental.pallas import tpu_sc as plsc

B, C, H, W = 32, 384, 14, 14
N = H * W
K = 9
OUT = 384
RED = 16
NP = 208

_F32 = jnp.float32


def _tc_kernel(x_ref, wcat_ref, b_ref, fc1_ref, fc2_ref, saw_ref, sab_ref,
               at_ref, bvt_ref, idx_ref):
    x = x_ref[0]

    avg = jnp.mean(x, axis=1, keepdims=True)
    mx = jnp.max(x, axis=1, keepdims=True)
    fc1 = fc1_ref[...]
    fc2 = fc2_ref[...]

    def mlp(v):
        h = jnp.maximum(jnp.dot(fc1, v, preferred_element_type=_F32), 0.0)
        return jnp.dot(fc2, h, preferred_element_type=_F32)

    ca = jax.nn.sigmoid(mlp(avg) + mlp(mx))
    x1 = x * ca

    m_mean = jnp.mean(x1, axis=0, keepdims=True)
    m_max = jnp.max(x1, axis=0, keepdims=True)
    sa_in = jnp.concatenate([m_mean, m_max], axis=0)
    PAD = 45
    padded = jnp.concatenate(
        [jnp.zeros((2, PAD), _F32), sa_in, jnp.zeros((2, PAD), _F32)], axis=1)
    px = jax.lax.broadcasted_iota(jnp.int32, (1, N), 1) % W
    saw = saw_ref[...]
    acc = jnp.zeros((1, N), _F32)
    for dy in range(-3, 4):
        for dx in range(-3, 4):
            s = dy * W + dx
            sh = jax.lax.slice(padded, (0, PAD + s), (2, PAD + s + N))
            wcol = jax.lax.slice(saw, (0, (dy + 3) * 7 + (dx + 3)),
                                 (2, (dy + 3) * 7 + (dx + 3) + 1))
            msk = ((px + dx >= 0) & (px + dx < W)).astype(_F32)
            acc = acc + jnp.sum(sh * wcol, axis=0, keepdims=True) * msk
    sa = acc + sab_ref[...]
    att = x1 * jax.nn.sigmoid(sa)

    amax = jnp.max(att, axis=1, keepdims=True)
    e = jnp.exp(att - amax)
    soft = e / jnp.sum(e, axis=1, keepdims=True)
    smax = jnp.max(soft, axis=1, keepdims=True)
    att = soft / (smax + 1e-10)
    att = (2.0 * att - 1.0) / 40.0 + 1.0
    xa = x * att

    nrm = jnp.sqrt(jnp.sum(x * x, axis=0, keepdims=True))
    v = x / (nrm + 1e-12)
    vT = v.T
    sq_col = jnp.sum(vT * vT, axis=1, keepdims=True)
    g = jnp.dot(vT, v, preferred_element_type=_F32)
    neg = 2.0 * g - sq_col - sq_col.T

    iota_l = jax.lax.broadcasted_iota(jnp.int32, (N, N), 1)
    idx_rows = []
    for _ in range(K):
        mval = jnp.max(neg, axis=1, keepdims=True)
        cand = jnp.where(neg == mval, iota_l, N)
        idxk = jnp.min(cand, axis=1, keepdims=True)
        idx_rows.append(idxk.T)
        neg = jnp.where(iota_l == idxk, -jnp.inf, neg)

    ab = jnp.dot(wcat_ref[...].astype(jnp.bfloat16), xa.astype(jnp.bfloat16),
                 preferred_element_type=_F32)
    a_part = jax.lax.slice(ab, (0, 0), (OUT, N)) + b_ref[...]
    b_part = jax.lax.slice(ab, (OUT, 0), (2 * OUT, N))

    zpad = jnp.zeros((NP - N, OUT), _F32)
    at_ref[0] = jnp.concatenate([a_part.T, zpad], axis=0)
    bvt_ref[0] = jnp.concatenate([b_part.T, zpad], axis=0)
    idx2 = jnp.concatenate(idx_rows, axis=0)
    idx_ref[0] = jnp.concatenate(
        [idx2, jnp.zeros((K, NP - N), jnp.int32)], axis=1)


def _tc_stage(xr, wcat, bcol, ca_fc1, ca_fc2, saw, sab):
    return pl.pallas_call(
        _tc_kernel,
        grid=(B,),
        in_specs=[
            pl.BlockSpec((1, C, N), lambda i: (i, 0, 0)),
            pl.BlockSpec((2 * OUT, C), lambda i: (0, 0)),
            pl.BlockSpec((OUT, 1), lambda i: (0, 0)),
            pl.BlockSpec((C // RED, C), lambda i: (0, 0)),
            pl.BlockSpec((C, C // RED), lambda i: (0, 0)),
            pl.BlockSpec((2, 49), lambda i: (0, 0)),
            pl.BlockSpec((1, 1), lambda i: (0, 0)),
        ],
        out_specs=[
            pl.BlockSpec((1, NP, OUT), lambda i: (i, 0, 0)),
            pl.BlockSpec((1, NP, OUT), lambda i: (i, 0, 0)),
            pl.BlockSpec((1, K, NP), lambda i: (i, 0, 0)),
        ],
        out_shape=[
            jax.ShapeDtypeStruct((B, NP, OUT), _F32),
            jax.ShapeDtypeStruct((B, NP, OUT), _F32),
            jax.ShapeDtypeStruct((B, K, NP), jnp.int32),
        ],
    )(xr, wcat, bcol, ca_fc1, ca_fc2, saw, sab)


_NC = 2
_GROUPS = NP // 16


@functools.partial(
    pl.kernel,
    mesh=plsc.VectorSubcoreMesh(core_axis_name="c", subcore_axis_name="s"),
    compiler_params=pltpu.CompilerParams(use_tc_tiling_on_sc=False,
                                         needs_layout_passes=False),
    out_type=jax.ShapeDtypeStruct((B * NP * OUT,), _F32),
    scratch_types=[
        pltpu.VMEM((NP * OUT,), _F32),
        pltpu.VMEM((K * NP,), jnp.int32),
        pltpu.VMEM((16 * OUT,), _F32),
        pltpu.VMEM((16 * OUT,), _F32),
    ],
)
def _sc_gather_max(bvt_hbm, at_hbm, idx_hbm, out_hbm, slab, idxv, atg, gbuf):
    wid = lax.axis_index("s") * _NC + lax.axis_index("c")
    ebase = wid * NP * OUT
    pltpu.sync_copy(bvt_hbm.at[pl.ds(ebase, NP * OUT)], slab)
    pltpu.sync_copy(idx_hbm.at[pl.ds(wid * K * NP, K * NP)], idxv)
    lanes = lax.iota(jnp.int32, 16)

    def group_body(gr, carry):
        gbase = gr * 16 * OUT
        pltpu.sync_copy(at_hbm.at[pl.ds(ebase + gbase, 16 * OUT)], atg)

        def chan_body(ch, carry2):
            chv = jnp.broadcast_to(ch, (16,))
            rows0 = idxv[pl.ds(gr * 16, 16)]
            m = plsc.load_gather(slab, [rows0 * OUT + chv])
            for k in range(1, K):
                rows = idxv[pl.ds(k * NP + gr * 16, 16)]
                m = jnp.maximum(m, plsc.load_gather(slab, [rows * OUT + chv]))
            flat = lanes * OUT + chv
            a = plsc.load_gather(atg, [flat])
            o = jnp.maximum(a + m, 0.0)
            plsc.store_scatter(gbuf, [flat], o)
            return carry2

        lax.fori_loop(0, OUT, chan_body, 0, unroll=4)
        pltpu.sync_copy(gbuf, out_hbm.at[pl.ds(ebase + gbase, 16 * OUT)])
        return carry

    lax.fori_loop(0, _GROUPS, group_body, 0)


def kernel(x, conv_w, conv_b, ca_fc1, ca_fc2, sa_w, sa_b):
    xr = x.reshape(B, C, N)
    w1 = conv_w[:, :C]
    w2 = conv_w[:, C:]
    wcat = jnp.concatenate([w1 - w2, w2], axis=0)
    bcol = conv_b.reshape(OUT, 1)
    saw = sa_w.reshape(2, 49)
    sab = sa_b.reshape(1, 1)

    at, bvt, idx = _tc_stage(xr, wcat, bcol, ca_fc1, ca_fc2, saw, sab)
    out_t = _sc_gather_max(bvt.reshape(-1), at.reshape(-1), idx.reshape(-1))
    out = out_t.reshape(B, NP, OUT)[:, :N, :]
    return out.transpose(0, 2, 1).reshape(B, OUT, H, W)


# speedup vs baseline: 2.3338x; 2.3338x over previous
import functools

import jax
import jax.numpy as jnp
from jax import lax
from jax.experimental import pallas as pl
from jax.experimental.pallas import tpu as pltpu
from jax.experimental.pallas import tpu_sc as plsc

B, C, H, W = 32, 384, 14, 14
N = H * W
K = 9
OUT = 384
RED = 16
NP = 208

_F32 = jnp.float32


def _tc_kernel(x_ref, wcat_ref, b_ref, fc1_ref, fc2_ref, saw_ref, sab_ref,
               at_ref, bvt_ref, idx_ref):
    x = x_ref[0]

    avg = jnp.mean(x, axis=1, keepdims=True)
    mx = jnp.max(x, axis=1, keepdims=True)
    fc1 = fc1_ref[...]
    fc2 = fc2_ref[...]

    def mlp(v):
        h = jnp.maximum(jnp.dot(fc1, v, preferred_element_type=_F32), 0.0)
        return jnp.dot(fc2, h, preferred_element_type=_F32)

    ca = jax.nn.sigmoid(mlp(avg) + mlp(mx))
    x1 = x * ca

    m_mean = jnp.mean(x1, axis=0, keepdims=True)
    m_max = jnp.max(x1, axis=0, keepdims=True)
    sa_in = jnp.concatenate([m_mean, m_max], axis=0)
    PAD = 45
    padded = jnp.concatenate(
        [jnp.zeros((2, PAD), _F32), sa_in, jnp.zeros((2, PAD), _F32)], axis=1)
    px = jax.lax.broadcasted_iota(jnp.int32, (1, N), 1) % W
    saw = saw_ref[...]
    acc = jnp.zeros((1, N), _F32)
    for dy in range(-3, 4):
        for dx in range(-3, 4):
            s = dy * W + dx
            sh = jax.lax.slice(padded, (0, PAD + s), (2, PAD + s + N))
            wcol = jax.lax.slice(saw, (0, (dy + 3) * 7 + (dx + 3)),
                                 (2, (dy + 3) * 7 + (dx + 3) + 1))
            msk = ((px + dx >= 0) & (px + dx < W)).astype(_F32)
            acc = acc + jnp.sum(sh * wcol, axis=0, keepdims=True) * msk
    sa = acc + sab_ref[...]
    att = x1 * jax.nn.sigmoid(sa)

    amax = jnp.max(att, axis=1, keepdims=True)
    e = jnp.exp(att - amax)
    soft = e / jnp.sum(e, axis=1, keepdims=True)
    smax = jnp.max(soft, axis=1, keepdims=True)
    att = soft / (smax + 1e-10)
    att = (2.0 * att - 1.0) / 40.0 + 1.0
    xa = x * att

    nrm = jnp.sqrt(jnp.sum(x * x, axis=0, keepdims=True))
    v = x / (nrm + 1e-12)
    vT = v.T
    sq_col = jnp.sum(vT * vT, axis=1, keepdims=True)
    g = jnp.dot(vT, v, preferred_element_type=_F32)
    neg = 2.0 * g - sq_col - sq_col.T

    iota_l = jax.lax.broadcasted_iota(jnp.int32, (N, N), 1)
    idx_rows = []
    for _ in range(K):
        mval = jnp.max(neg, axis=1, keepdims=True)
        cand = jnp.where(neg == mval, iota_l, N)
        idxk = jnp.min(cand, axis=1, keepdims=True)
        idx_rows.append(idxk.T)
        neg = jnp.where(iota_l == idxk, -jnp.inf, neg)

    ab = jnp.dot(wcat_ref[...].astype(jnp.bfloat16), xa.astype(jnp.bfloat16),
                 preferred_element_type=_F32)
    a_part = jax.lax.slice(ab, (0, 0), (OUT, N)) + b_ref[...]
    b_part = jax.lax.slice(ab, (OUT, 0), (2 * OUT, N))

    zpad = jnp.zeros((OUT, NP - N), _F32)
    at_ref[0] = jnp.concatenate([a_part, zpad], axis=1)
    bvt_ref[0] = jnp.concatenate([b_part, zpad], axis=1)
    idx2 = jnp.concatenate(idx_rows, axis=0)
    idx_ref[0] = jnp.concatenate(
        [idx2, jnp.zeros((K, NP - N), jnp.int32)], axis=1)


def _tc_stage(xr, wcat, bcol, ca_fc1, ca_fc2, saw, sab):
    return pl.pallas_call(
        _tc_kernel,
        grid=(B,),
        in_specs=[
            pl.BlockSpec((1, C, N), lambda i: (i, 0, 0)),
            pl.BlockSpec((2 * OUT, C), lambda i: (0, 0)),
            pl.BlockSpec((OUT, 1), lambda i: (0, 0)),
            pl.BlockSpec((C // RED, C), lambda i: (0, 0)),
            pl.BlockSpec((C, C // RED), lambda i: (0, 0)),
            pl.BlockSpec((2, 49), lambda i: (0, 0)),
            pl.BlockSpec((1, 1), lambda i: (0, 0)),
        ],
        out_specs=[
            pl.BlockSpec((1, OUT, NP), lambda i: (i, 0, 0)),
            pl.BlockSpec((1, OUT, NP), lambda i: (i, 0, 0)),
            pl.BlockSpec((1, K, NP), lambda i: (i, 0, 0)),
        ],
        out_shape=[
            jax.ShapeDtypeStruct((B, OUT, NP), _F32),
            jax.ShapeDtypeStruct((B, OUT, NP), _F32),
            jax.ShapeDtypeStruct((B, K, NP), jnp.int32),
        ],
    )(xr, wcat, bcol, ca_fc1, ca_fc2, saw, sab)


_NC = 2
_GROUPS = NP // 16


@functools.partial(
    pl.kernel,
    mesh=plsc.VectorSubcoreMesh(core_axis_name="c", subcore_axis_name="s"),
    compiler_params=pltpu.CompilerParams(use_tc_tiling_on_sc=False,
                                         needs_layout_passes=False),
    out_type=jax.ShapeDtypeStruct((B, OUT, NP), _F32),
    scratch_types=[
        pltpu.VMEM((OUT * NP,), _F32),
        pltpu.VMEM((K * NP,), jnp.int32),
        pltpu.VMEM((OUT, 16), _F32),
        pltpu.VMEM((OUT, 16), _F32),
    ],
)
def _sc_gather_max(bvt_hbm, at_hbm, idx_hbm, out_hbm, slab, idxv, atg, gbuf):
    wid = lax.axis_index("s") * _NC + lax.axis_index("c")
    pltpu.sync_copy(bvt_hbm.at[wid], slab)
    pltpu.sync_copy(idx_hbm.at[wid], idxv)

    def group_body(gr, carry):
        pltpu.sync_copy(at_hbm.at[wid, :, pl.ds(gr * 16, 16)], atg)
        rows = [idxv[pl.ds(k * NP + gr * 16, 16)] for k in range(K)]

        @plsc.parallel_loop(0, OUT, unroll=8)
        def chan_body(ch):
            base = ch * NP
            m = plsc.load_gather(slab, [rows[0] + base])
            for k in range(1, K):
                m = jnp.maximum(m, plsc.load_gather(slab, [rows[k] + base]))
            o = jnp.maximum(atg[ch] + m, 0.0)
            gbuf[ch] = o

        pltpu.sync_copy(gbuf, out_hbm.at[wid, :, pl.ds(gr * 16, 16)])
        return carry

    lax.fori_loop(0, _GROUPS, group_body, 0)


def kernel(x, conv_w, conv_b, ca_fc1, ca_fc2, sa_w, sa_b):
    xr = x.reshape(B, C, N)
    w1 = conv_w[:, :C]
    w2 = conv_w[:, C:]
    wcat = jnp.concatenate([w1 - w2, w2], axis=0)
    bcol = conv_b.reshape(OUT, 1)
    saw = sa_w.reshape(2, 49)
    sab = sa_b.reshape(1, 1)

    at, bvt, idx = _tc_stage(xr, wcat, bcol, ca_fc1, ca_fc2, saw, sab)
    out_p = _sc_gather_max(bvt.reshape(B, OUT * NP), at, idx.reshape(B, K * NP))
    return out_p[:, :, :N].reshape(B, OUT, H, W)
